# baseline (speedup 1.0000x reference)
import functools

import jax
import jax.numpy as jnp
from jax import lax
from jax.experimental import pallas as pl
from jax.experimental.pallas import tpu as pltpu
from jax.experimental.pallas import tpu_sc as plsc

B, N = 64, 8192
NCHUNK = 4
KSEL = 256
L = 16
NC, NS = 2, 16
NW = NC * NS
ROWS_PER_W = B // NW
NV = N // L
CV = NV // NCHUNK
CN = N // NCHUNK
NBINS = 256
NBV = NBINS // L
MSB = -2147483648


def _splat(v, dtype=jnp.int32):
    return jnp.full((L,), v, dtype)


def _thresholds(x_ref, cbuf_ref, hists, cntges, wait_chunk):
    ones = _splat(1)
    zeros = _splat(0)
    offs = tuple(r * N for r in range(ROWS_PER_W))

    def biased_key(sl):
        b = lax.bitcast_convert_type(x_ref[sl], jnp.int32)
        flip = jnp.right_shift(b, _splat(31)) | _splat(MSB)
        return b ^ flip

    def zero_hists():
        @plsc.parallel_loop(0, NBV, unroll=4)
        def _(j):
            for h in hists:
                h[pl.ds(j * L, L)] = zeros

    def select(krems):
        def sb(j, carry):
            sufs, nms = carry
            i = (NBV - 1) - j
            sl = pl.ds(i * L, L)
            new_sufs, new_nms = [], []
            for r in range(ROWS_PER_W):
                h = hists[r][sl]
                hrev = lax.rev(h, (0,))
                cgrev = plsc.cumsum(hrev) + sufs[r]
                cg = lax.rev(cgrev, (0,))
                cntges[r][sl] = cg
                new_nms.append(
                    nms[r] + plsc.all_reduce_population_count(cg >= krems[r]))
                new_sufs.append(sufs[r] + jnp.broadcast_to(jnp.sum(h), (L,)))
            return tuple(new_sufs), tuple(new_nms)

        init = ((zeros,) * ROWS_PER_W, (zeros,) * ROWS_PER_W)
        _, nms = lax.fori_loop(0, NBV, sb, init)
        out = []
        for r in range(ROWS_PER_W):
            tstar = nms[r] - ones
            h_t = plsc.load_gather(hists[r], [tstar])
            cg_t = plsc.load_gather(cntges[r], [tstar])
            out.append((tstar, cg_t - h_t))
        return out

    zero_hists()

    for c in range(NCHUNK):
        wait_chunk(c)

        @plsc.parallel_loop(c * CV, (c + 1) * CV, unroll=4)
        def _(i):
            for r in range(ROWS_PER_W):
                u = biased_key(pl.ds(offs[r] + i * L, L))
                bucket = lax.shift_right_logical(u, _splat(24))
                plsc.addupdate_scatter(hists[r], [bucket], ones)

    krems = [_splat(KSEL)] * ROWS_PER_W
    sel = select(krems)
    prefixes = [s[0] for s in sel]
    krems = [krems[r] - sel[r][1] for r in range(ROWS_PER_W)]

    zero_hists()

    def p1(i, cnts, prefixes=tuple(prefixes)):
        new_cnts = []
        for r in range(ROWS_PER_W):
            u = biased_key(pl.ds(offs[r] + i * L, L))
            pref = lax.shift_right_logical(u, _splat(24))
            m = pref == prefixes[r]
            bucket = lax.shift_right_logical(u, _splat(16)) & _splat(0xFF)
            plsc.addupdate_scatter(hists[r], [bucket], ones, mask=m)
            pos = plsc.cumsum(m.astype(jnp.int32)) - ones + cnts[r]
            plsc.store_scatter(cbuf_ref.at[pl.ds(offs[r], N)], [pos], u,
                               mask=m)
            new_cnts.append(cnts[r] + plsc.all_reduce_population_count(m))
        return tuple(new_cnts)

    cnts = plsc.parallel_loop(0, NV, unroll=4,
                              carry=(zeros,) * ROWS_PER_W)(p1)
    sel = select(krems)
    prefixes = [
        lax.shift_left(prefixes[r], _splat(8)) | sel[r][0]
        for r in range(ROWS_PER_W)
    ]
    krems = [krems[r] - sel[r][1] for r in range(ROWS_PER_W)]

    maxcnt = jnp.max(jnp.maximum(cnts[0], cnts[1]))
    nvc = jnp.right_shift(maxcnt + 15, 4)
    lane = jnp.arange(L, dtype=jnp.int32)
    for p in range(2, 4):
        shift = 24 - 8 * p

        zero_hists()

        def hp(i, shift=shift, prefixes=tuple(prefixes)):
            for r in range(ROWS_PER_W):
                u = cbuf_ref[pl.ds(offs[r] + i * L, L)]
                valid = (lane + i * L) < cnts[r]
                pref = lax.shift_right_logical(u, _splat(shift + 8))
                m = valid & (pref == prefixes[r])
                bucket = (lax.shift_right_logical(u, _splat(shift))
                          & _splat(0xFF))
                plsc.addupdate_scatter(hists[r], [bucket], ones, mask=m)

        plsc.parallel_loop(0, nvc, unroll=4)(hp)
        sel = select(krems)
        prefixes = [
            lax.shift_left(prefixes[r], _splat(8)) | sel[r][0]
            for r in range(ROWS_PER_W)
        ]
        krems = [krems[r] - sel[r][1] for r in range(ROWS_PER_W)]

    thrs = []
    for r in range(ROWS_PER_W):
        u = prefixes[r]
        flip2 = jnp.right_shift(~u, _splat(31)) | _splat(MSB)
        thrs.append(lax.bitcast_convert_type(u ^ flip2, jnp.float32))
    return thrs


_MESH = plsc.VectorSubcoreMesh(core_axis_name="c", subcore_axis_name="s")


@functools.partial(
    pl.kernel,
    out_type=jax.ShapeDtypeStruct((B, N), jnp.float32),
    mesh=_MESH,
    compiler_params=pltpu.CompilerParams(needs_layout_passes=False),
    scratch_types=[
        pltpu.VMEM((ROWS_PER_W * N,), jnp.float32),
        pltpu.VMEM((ROWS_PER_W * N,), jnp.int32),
        pltpu.VMEM((NBINS,), jnp.int32),
        pltpu.VMEM((NBINS,), jnp.int32),
        pltpu.VMEM((NBINS,), jnp.int32),
        pltpu.VMEM((NBINS,), jnp.int32),
        pltpu.SemaphoreType.DMA((NCHUNK,)),
        pltpu.SemaphoreType.DMA((NCHUNK,)),
    ],
)
def _sparsify(x_hbm, out_hbm, x_v, cbuf_v, hist0, hist1, cg0, cg1, sem_in,
              sem_out):
    wid = lax.axis_index("s") * NC + lax.axis_index("c")
    base = wid * ROWS_PER_W
    in_copies = [
        [
            pltpu.async_copy(
                x_hbm.at[base + r, pl.ds(c * CN, CN)],
                x_v.at[pl.ds(r * N + c * CN, CN)],
                sem_in.at[c],
            )
            for r in range(ROWS_PER_W)
        ]
        for c in range(NCHUNK)
    ]

    def wait_chunk(c):
        for cp in in_copies[c]:
            cp.wait()

    fzero = _splat(0.0, jnp.float32)
    thrs = _thresholds(x_v, cbuf_v, (hist0, hist1), (cg0, cg1), wait_chunk)

    out_copies = []
    for c in range(NCHUNK):
        @plsc.parallel_loop(c * CV, (c + 1) * CV, unroll=4)
        def _(i):
            for r in range(ROWS_PER_W):
                sl = pl.ds(r * N + i * L, L)
                xv = x_v[sl]
                x_v[sl] = jnp.where(xv >= thrs[r], xv, fzero)

        out_copies += [
            pltpu.async_copy(
                x_v.at[pl.ds(r * N + c * CN, CN)],
                out_hbm.at[base + r, pl.ds(c * CN, CN)],
                sem_out.at[c],
            )
            for r in range(ROWS_PER_W)
        ]
    for cp in out_copies:
        cp.wait()


def kernel(x):
    return _sparsify(x)


# speedup vs baseline: 1.0203x; 1.0203x over previous
import functools

import jax
import jax.numpy as jnp
from jax import lax
from jax.experimental import pallas as pl
from jax.experimental.pallas import tpu as pltpu
from jax.experimental.pallas import tpu_sc as plsc

B, N = 64, 8192
NCHUNK = 2
KSEL = 256
L = 16
NC, NS = 2, 16
NW = NC * NS
ROWS_PER_W = B // NW
NV = N // L
CV = NV // NCHUNK
CN = N // NCHUNK
NBINS = 256
NBV = NBINS // L
MSB = -2147483648


def _splat(v, dtype=jnp.int32):
    return jnp.full((L,), v, dtype)


def _thresholds(x_ref, cbuf_ref, hists, cntges, wait_chunk):
    ones = _splat(1)
    zeros = _splat(0)
    offs = tuple(r * N for r in range(ROWS_PER_W))

    def biased_key(sl):
        b = lax.bitcast_convert_type(x_ref[sl], jnp.int32)
        flip = jnp.right_shift(b, _splat(31)) | _splat(MSB)
        return b ^ flip

    def zero_hists():
        @plsc.parallel_loop(0, NBV, unroll=4)
        def _(j):
            for h in hists:
                h[pl.ds(j * L, L)] = zeros

    def select(krems):
        def sb(j, carry):
            sufs, nms = carry
            i = (NBV - 1) - j
            sl = pl.ds(i * L, L)
            new_sufs, new_nms = [], []
            for r in range(ROWS_PER_W):
                h = hists[r][sl]
                hrev = lax.rev(h, (0,))
                cgrev = plsc.cumsum(hrev) + sufs[r]
                cg = lax.rev(cgrev, (0,))
                cntges[r][sl] = cg
                new_nms.append(
                    nms[r] + plsc.all_reduce_population_count(cg >= krems[r]))
                new_sufs.append(sufs[r] + jnp.broadcast_to(jnp.sum(h), (L,)))
            return tuple(new_sufs), tuple(new_nms)

        init = ((zeros,) * ROWS_PER_W, (zeros,) * ROWS_PER_W)
        _, nms = lax.fori_loop(0, NBV, sb, init)
        out = []
        for r in range(ROWS_PER_W):
            tstar = nms[r] - ones
            h_t = plsc.load_gather(hists[r], [tstar])
            cg_t = plsc.load_gather(cntges[r], [tstar])
            out.append((tstar, cg_t - h_t))
        return out

    zero_hists()

    for c in range(NCHUNK):
        wait_chunk(c)

        @plsc.parallel_loop(c * CV, (c + 1) * CV, unroll=4)
        def _(i):
            for r in range(ROWS_PER_W):
                u = biased_key(pl.ds(offs[r] + i * L, L))
                bucket = lax.shift_right_logical(u, _splat(24))
                plsc.addupdate_scatter(hists[r], [bucket], ones)

    krems = [_splat(KSEL)] * ROWS_PER_W
    sel = select(krems)
    prefixes = [s[0] for s in sel]
    krems = [krems[r] - sel[r][1] for r in range(ROWS_PER_W)]

    zero_hists()

    def p1(i, cnts, prefixes=tuple(prefixes)):
        new_cnts = []
        for r in range(ROWS_PER_W):
            u = biased_key(pl.ds(offs[r] + i * L, L))
            pref = lax.shift_right_logical(u, _splat(24))
            m = pref == prefixes[r]
            bucket = lax.shift_right_logical(u, _splat(16)) & _splat(0xFF)
            plsc.addupdate_scatter(hists[r], [bucket], ones, mask=m)
            pos = plsc.cumsum(m.astype(jnp.int32)) - ones + cnts[r]
            plsc.store_scatter(cbuf_ref.at[pl.ds(offs[r], N)], [pos], u,
                               mask=m)
            new_cnts.append(cnts[r] + plsc.all_reduce_population_count(m))
        return tuple(new_cnts)

    cnts = plsc.parallel_loop(0, NV, unroll=4,
                              carry=(zeros,) * ROWS_PER_W)(p1)
    sel = select(krems)
    prefixes = [
        lax.shift_left(prefixes[r], _splat(8)) | sel[r][0]
        for r in range(ROWS_PER_W)
    ]
    krems = [krems[r] - sel[r][1] for r in range(ROWS_PER_W)]

    maxcnt = jnp.max(jnp.maximum(cnts[0], cnts[1]))
    nvc = jnp.right_shift(maxcnt + 15, 4)
    lane = jnp.arange(L, dtype=jnp.int32)
    for p in range(2, 4):
        shift = 24 - 8 * p

        zero_hists()

        def hp(i, shift=shift, prefixes=tuple(prefixes)):
            for r in range(ROWS_PER_W):
                u = cbuf_ref[pl.ds(offs[r] + i * L, L)]
                valid = (lane + i * L) < cnts[r]
                pref = lax.shift_right_logical(u, _splat(shift + 8))
                m = valid & (pref == prefixes[r])
                bucket = (lax.shift_right_logical(u, _splat(shift))
                          & _splat(0xFF))
                plsc.addupdate_scatter(hists[r], [bucket], ones, mask=m)

        plsc.parallel_loop(0, nvc, unroll=4)(hp)
        sel = select(krems)
        prefixes = [
            lax.shift_left(prefixes[r], _splat(8)) | sel[r][0]
            for r in range(ROWS_PER_W)
        ]
        krems = [krems[r] - sel[r][1] for r in range(ROWS_PER_W)]

    thrs = []
    for r in range(ROWS_PER_W):
        u = prefixes[r]
        flip2 = jnp.right_shift(~u, _splat(31)) | _splat(MSB)
        thrs.append(lax.bitcast_convert_type(u ^ flip2, jnp.float32))
    return thrs


_MESH = plsc.VectorSubcoreMesh(core_axis_name="c", subcore_axis_name="s")


@functools.partial(
    pl.kernel,
    out_type=jax.ShapeDtypeStruct((B, N), jnp.float32),
    mesh=_MESH,
    compiler_params=pltpu.CompilerParams(needs_layout_passes=False),
    scratch_types=[
        pltpu.VMEM((ROWS_PER_W * N,), jnp.float32),
        pltpu.VMEM((ROWS_PER_W * N,), jnp.int32),
        pltpu.VMEM((NBINS,), jnp.int32),
        pltpu.VMEM((NBINS,), jnp.int32),
        pltpu.VMEM((NBINS,), jnp.int32),
        pltpu.VMEM((NBINS,), jnp.int32),
        pltpu.SemaphoreType.DMA((NCHUNK,)),
        pltpu.SemaphoreType.DMA((NCHUNK,)),
    ],
)
def _sparsify(x_hbm, out_hbm, x_v, cbuf_v, hist0, hist1, cg0, cg1, sem_in,
              sem_out):
    wid = lax.axis_index("s") * NC + lax.axis_index("c")
    base = wid * ROWS_PER_W
    in_copies = [
        [
            pltpu.async_copy(
                x_hbm.at[base + r, pl.ds(c * CN, CN)],
                x_v.at[pl.ds(r * N + c * CN, CN)],
                sem_in.at[c],
            )
            for r in range(ROWS_PER_W)
        ]
        for c in range(NCHUNK)
    ]

    def wait_chunk(c):
        for cp in in_copies[c]:
            cp.wait()

    fzero = _splat(0.0, jnp.float32)
    thrs = _thresholds(x_v, cbuf_v, (hist0, hist1), (cg0, cg1), wait_chunk)

    out_copies = []
    for c in range(NCHUNK):
        @plsc.parallel_loop(c * CV, (c + 1) * CV, unroll=4)
        def _(i):
            for r in range(ROWS_PER_W):
                sl = pl.ds(r * N + i * L, L)
                xv = x_v[sl]
                x_v[sl] = jnp.where(xv >= thrs[r], xv, fzero)

        out_copies += [
            pltpu.async_copy(
                x_v.at[pl.ds(r * N + c * CN, CN)],
                out_hbm.at[base + r, pl.ds(c * CN, CN)],
                sem_out.at[c],
            )
            for r in range(ROWS_PER_W)
        ]
    for cp in out_copies:
        cp.wait()


def kernel(x):
    return _sparsify(x)
